# baseline (speedup 1.0000x reference)
import jax
import jax.numpy as jnp
from jax.experimental import pallas as pl
from jax.experimental.pallas import tpu as pltpu

_B = 128
_L = 512
_NSEG = 50
_VOCAB = 3001
_BB = 16
_NCOPY = _B // _BB


def _fill_kernel(out_hbm, mask_hbm, ib_hbm, pat, spat, ones_m, ones_ib, sems):
    pat[...] = jnp.zeros(pat.shape, jnp.float32)
    pat[:, :, pl.ds(0, 1)] = jnp.ones((_BB, _NSEG, 1), jnp.float32)
    col = jax.lax.broadcasted_iota(jnp.int32, (1, _NSEG, _VOCAB), 2)
    seg = jax.lax.broadcasted_iota(jnp.int32, (1, _NSEG, _VOCAB), 1)
    spat[...] = (col == jnp.where(seg == 0, 1, 0)).astype(jnp.float32)
    ones_m[...] = jnp.ones(ones_m.shape, jnp.float32)
    ones_ib[...] = jnp.ones(ones_ib.shape, jnp.float32)

    copies = [
        pltpu.make_async_copy(spat, out_hbm.at[pl.ds(0, 1)], sems.at[0]),
        pltpu.make_async_copy(
            pat.at[pl.ds(0, _BB - 1)], out_hbm.at[pl.ds(1, _BB - 1)], sems.at[1]),
        pltpu.make_async_copy(ones_m, mask_hbm, sems.at[2]),
        pltpu.make_async_copy(ones_ib, ib_hbm, sems.at[3]),
    ]
    for i in range(1, _NCOPY):
        copies.append(pltpu.make_async_copy(
            pat, out_hbm.at[pl.ds(_BB * i, _BB)], sems.at[3 + i]))
    for c in copies:
        c.start()
    for c in copies:
        c.wait()


def kernel(x):
    del x
    out, mask, in_boundary = pl.pallas_call(
        _fill_kernel,
        out_specs=[
            pl.BlockSpec(memory_space=pltpu.MemorySpace.HBM),
            pl.BlockSpec(memory_space=pltpu.MemorySpace.HBM),
            pl.BlockSpec(memory_space=pltpu.MemorySpace.HBM),
        ],
        out_shape=[
            jax.ShapeDtypeStruct((_B, _NSEG, _VOCAB), jnp.float32),
            jax.ShapeDtypeStruct((_B, _NSEG), jnp.float32),
            jax.ShapeDtypeStruct((_B, _L + 1), jnp.float32),
        ],
        scratch_shapes=[
            pltpu.VMEM((_BB, _NSEG, _VOCAB), jnp.float32),
            pltpu.VMEM((1, _NSEG, _VOCAB), jnp.float32),
            pltpu.VMEM((_B, _NSEG), jnp.float32),
            pltpu.VMEM((_B, _L + 1), jnp.float32),
            pltpu.SemaphoreType.DMA((4 + _NCOPY,)),
        ],
    )()
    return (out, mask, in_boundary)
